# baseline (speedup 1.0000x reference)
import functools

import jax
import jax.numpy as jnp
from jax import lax
from jax.experimental import pallas as pl
from jax.experimental.pallas import tpu as pltpu
from jax.experimental.pallas import tpu_sc as plsc

B = 16384
D = 128
L = 1000
LP = 1024
NC = 2
NS = 16
NW = NC * NS
BPW = B // NW
C = 128
NCHUNK = BPW // C
LANES = 16


def kernel(row, col, dir, row_emb, col_emb, dir_emb):
    colp = jnp.concatenate(
        [col_emb, jnp.zeros((LP - L, D), jnp.float32)], axis=0)
    row2 = row.astype(jnp.int32).reshape(B // C, C)
    col2 = col.astype(jnp.int32).reshape(B // C, C)
    dir2 = dir.astype(jnp.int32).reshape(B // C, C)

    mesh = plsc.VectorSubcoreMesh(core_axis_name="c", subcore_axis_name="s")

    @functools.partial(
        pl.kernel,
        mesh=mesh,
        out_type=jax.ShapeDtypeStruct((B, D), jnp.float32),
        scratch_types=[
            pltpu.VMEM((NCHUNK, C), jnp.int32),
            pltpu.VMEM((NCHUNK, C), jnp.int32),
            pltpu.VMEM((NCHUNK, C), jnp.int32),
            pltpu.VMEM((NCHUNK, C), jnp.int32),
            pltpu.VMEM((2, D), jnp.float32),
            pltpu.VMEM((C, D), jnp.float32),
            pltpu.VMEM((C, D), jnp.float32),
            pltpu.VMEM((C, D), jnp.float32),
            pltpu.VMEM((C, D), jnp.float32),
            pltpu.VMEM((C, D), jnp.float32),
            pltpu.VMEM((C, D), jnp.float32),
            pltpu.VMEM_SHARED((2 * LP, D), jnp.float32),
            pltpu.SemaphoreType.DMA,
            pltpu.SemaphoreType.DMA,
            pltpu.SemaphoreType.DMA,
        ],
    )
    def k(row_hbm, col_hbm, dir_hbm, colp_hbm, de_hbm, re_hbm, out_hbm,
          ri_v, ci_v, di_v, cd_v, dd_v, a0_v, a1_v, b0_v, b1_v, o0_v, o1_v,
          sh_cd, asem, bsem, wsem):
        cid = lax.axis_index("c")
        sid = lax.axis_index("s")
        wid = sid * NC + cid
        chunk0 = wid * NCHUNK

        abufs = (a0_v, a1_v)
        bbufs = (b0_v, b1_v)
        obufs = (o0_v, o1_v)

        pltpu.sync_copy(row_hbm.at[pl.ds(chunk0, NCHUNK)], ri_v)
        ga0 = pltpu.async_copy(re_hbm.at[ri_v.at[0]], a0_v, asem)
        ga1 = pltpu.async_copy(re_hbm.at[ri_v.at[1]], a1_v, asem)

        d = sid // 8
        c0 = (sid % 8) * C
        pltpu.sync_copy(colp_hbm.at[pl.ds(c0, C)], o0_v)
        pltpu.sync_copy(de_hbm, dd_v)

        @pl.loop(0, C)
        def _(r):
            for j in range(0, D, LANES):
                s = pl.ds(j, LANES)
                o0_v[r, s] = o0_v[r, s] + dd_v[d, s]

        pltpu.sync_copy(o0_v, sh_cd.at[pl.ds(sid * C, C)])

        pltpu.sync_copy(col_hbm.at[pl.ds(chunk0, NCHUNK)], ci_v)
        pltpu.sync_copy(dir_hbm.at[pl.ds(chunk0, NCHUNK)], di_v)

        @pl.loop(0, NCHUNK)
        def _(r):
            for j in range(0, C, LANES):
                s = pl.ds(j, LANES)
                cd_v[r, s] = di_v[r, s] * LP + ci_v[r, s]

        plsc.subcore_barrier()

        gb0 = pltpu.async_copy(sh_cd.at[cd_v.at[0]], b0_v, bsem)
        gb1 = pltpu.async_copy(sh_cd.at[cd_v.at[1]], b1_v, bsem)

        def fire(ch):
            a = pltpu.async_copy(re_hbm.at[ri_v.at[ch]], abufs[ch % 2], asem)
            b = pltpu.async_copy(sh_cd.at[cd_v.at[ch]], bbufs[ch % 2], bsem)
            return a, b

        H = C // 2
        gcp = [(ga0, gb0), (ga1, gb1)]
        wcp = [None, None]
        for ch in range(NCHUNK):
            gcp[ch % 2][0].wait()
            gcp[ch % 2][1].wait()
            if wcp[ch % 2] is not None:
                wcp[ch % 2][0].wait()
                wcp[ch % 2][1].wait()
            a_v, b_v, o_v = abufs[ch % 2], bbufs[ch % 2], obufs[ch % 2]

            @pl.loop(0, H)
            def _(r):
                for j in range(0, D, LANES):
                    s = pl.ds(j, LANES)
                    o_v[r, s] = a_v[r, s] + b_v[r, s]

            w1 = pltpu.async_copy(
                o_v.at[pl.ds(0, H)],
                out_hbm.at[pl.ds((chunk0 + ch) * C, H)], wsem)

            @pl.loop(H, C)
            def _(r):
                for j in range(0, D, LANES):
                    s = pl.ds(j, LANES)
                    o_v[r, s] = a_v[r, s] + b_v[r, s]

            w2 = pltpu.async_copy(
                o_v.at[pl.ds(H, H)],
                out_hbm.at[pl.ds((chunk0 + ch) * C + H, H)], wsem)
            if ch + 2 < NCHUNK:
                gcp[ch % 2] = fire(ch + 2)
            wcp[ch % 2] = (w1, w2)
        wcp[0][0].wait()
        wcp[0][1].wait()
        wcp[1][0].wait()
        wcp[1][1].wait()

    return k(row2, col2, dir2, colp, dir_emb, row_emb)


# speedup vs baseline: 1.1642x; 1.1642x over previous
import functools

import jax
import jax.numpy as jnp
from jax import lax
from jax.experimental import pallas as pl
from jax.experimental.pallas import tpu as pltpu
from jax.experimental.pallas import tpu_sc as plsc

B = 16384
D = 128
L = 1000
TAB = 3072
NC = 2
NS = 16
NW = NC * NS
BPW = B // NW
C = 128
NCHUNK = BPW // C
LANES = 16
STG = TAB // NS


def _coldir_body(dir_ref, col_ref, o_ref):
    o_ref[...] = dir_ref[...][:, None, :] + col_ref[...][None, :, :]


def _build_coldir(col_emb, dir_emb):
    out = pl.pallas_call(
        _coldir_body,
        out_shape=jax.ShapeDtypeStruct((2, L, D), jnp.float32),
    )(dir_emb, col_emb)
    return out.reshape(2 * L, D)


def kernel(row, col, dir, row_emb, col_emb, dir_emb):
    coldir = (dir_emb[:, None, :] + col_emb[None, :, :]).reshape(2 * L, D)
    row2 = row.astype(jnp.int32).reshape(B // C, C)
    col2 = col.astype(jnp.int32).reshape(B // C, C)
    dir2 = dir.astype(jnp.int32).reshape(B // C, C)

    mesh = plsc.VectorSubcoreMesh(core_axis_name="c", subcore_axis_name="s")

    @functools.partial(
        pl.kernel,
        mesh=mesh,
        out_type=jax.ShapeDtypeStruct((B, D), jnp.float32),
        scratch_types=[
            pltpu.VMEM((NCHUNK, C), jnp.int32),
            pltpu.VMEM((NCHUNK, C), jnp.int32),
            pltpu.VMEM((NCHUNK, C), jnp.int32),
            pltpu.VMEM((NCHUNK, C), jnp.int32),
            pltpu.VMEM((C, D), jnp.float32),
            pltpu.VMEM((C, D), jnp.float32),
            pltpu.VMEM((C, D), jnp.float32),
            pltpu.VMEM((C, D), jnp.float32),
            pltpu.VMEM((C, D), jnp.float32),
            pltpu.VMEM((C, D), jnp.float32),
            pltpu.SemaphoreType.DMA,
            pltpu.SemaphoreType.DMA,
            pltpu.SemaphoreType.DMA,
        ],
    )
    def k(row_hbm, col_hbm, dir_hbm, cdtab_hbm, re_hbm, out_hbm,
          ri_v, ci_v, di_v, cd_v, a0_v, a1_v, b0_v, b1_v, o0_v, o1_v,
          asem, bsem, wsem):
        cid = lax.axis_index("c")
        sid = lax.axis_index("s")
        wid = sid * NC + cid
        chunk0 = wid * NCHUNK

        pltpu.sync_copy(row_hbm.at[pl.ds(chunk0, NCHUNK)], ri_v)
        abufs = (a0_v, a1_v)
        bbufs = (b0_v, b1_v)
        obufs = (o0_v, o1_v)

        ga0 = pltpu.async_copy(re_hbm.at[ri_v.at[0]], a0_v, asem)
        ga1 = pltpu.async_copy(re_hbm.at[ri_v.at[1]], a1_v, asem)

        pltpu.sync_copy(col_hbm.at[pl.ds(chunk0, NCHUNK)], ci_v)
        pltpu.sync_copy(dir_hbm.at[pl.ds(chunk0, NCHUNK)], di_v)

        @pl.loop(0, NCHUNK)
        def _(r):
            for j in range(0, C, LANES):
                s = pl.ds(j, LANES)
                cd_v[r, s] = di_v[r, s] * L + ci_v[r, s]

        gb0 = pltpu.async_copy(cdtab_hbm.at[cd_v.at[0]], b0_v, bsem)
        gb1 = pltpu.async_copy(cdtab_hbm.at[cd_v.at[1]], b1_v, bsem)

        def fire(ch):
            a = pltpu.async_copy(re_hbm.at[ri_v.at[ch]], abufs[ch % 2], asem)
            b = pltpu.async_copy(cdtab_hbm.at[cd_v.at[ch]], bbufs[ch % 2], bsem)
            return a, b

        H = C // 2
        gcp = [(ga0, gb0), (ga1, gb1)]
        wcp = [None, None]
        for ch in range(NCHUNK):
            gcp[ch % 2][0].wait()
            gcp[ch % 2][1].wait()
            if wcp[ch % 2] is not None:
                wcp[ch % 2][0].wait()
                wcp[ch % 2][1].wait()
            a_v, b_v, o_v = abufs[ch % 2], bbufs[ch % 2], obufs[ch % 2]

            @pl.loop(0, H)
            def _(r):
                for j in range(0, D, LANES):
                    s = pl.ds(j, LANES)
                    o_v[r, s] = a_v[r, s] + b_v[r, s]

            w1 = pltpu.async_copy(
                o_v.at[pl.ds(0, H)],
                out_hbm.at[pl.ds((chunk0 + ch) * C, H)], wsem)

            @pl.loop(H, C)
            def _(r):
                for j in range(0, D, LANES):
                    s = pl.ds(j, LANES)
                    o_v[r, s] = a_v[r, s] + b_v[r, s]

            w2 = pltpu.async_copy(
                o_v.at[pl.ds(H, H)],
                out_hbm.at[pl.ds((chunk0 + ch) * C + H, H)], wsem)
            if ch + 2 < NCHUNK:
                gcp[ch % 2] = fire(ch + 2)
            wcp[ch % 2] = (w1, w2)
        wcp[0][0].wait()
        wcp[0][1].wait()
        wcp[1][0].wait()
        wcp[1][1].wait()

    return k(row2, col2, dir2, coldir, row_emb)


# speedup vs baseline: 1.2909x; 1.1088x over previous
import functools

import jax
import jax.numpy as jnp
from jax import lax
from jax.experimental import pallas as pl
from jax.experimental.pallas import tpu as pltpu
from jax.experimental.pallas import tpu_sc as plsc

B = 16384
D = 128
L = 1000
TAB = 3072
NC = 2
NS = 16
NW = NC * NS
BPW = B // NW
C = 128
NCHUNK = BPW // C
LANES = 16
STG = TAB // NS


def kernel(row, col, dir, row_emb, col_emb, dir_emb):
    coldir = (dir_emb[:, None, :] + col_emb[None, :, :]).reshape(2 * L, D)
    tab = jnp.concatenate(
        [row_emb, coldir, jnp.zeros((TAB - 3 * L, D), jnp.float32)], axis=0)
    row2 = row.astype(jnp.int32).reshape(B // C, C)
    col2 = col.astype(jnp.int32).reshape(B // C, C)
    dir2 = dir.astype(jnp.int32).reshape(B // C, C)

    mesh = plsc.VectorSubcoreMesh(core_axis_name="c", subcore_axis_name="s")

    @functools.partial(
        pl.kernel,
        mesh=mesh,
        out_type=jax.ShapeDtypeStruct((B, D), jnp.float32),
        scratch_types=[
            pltpu.VMEM((NCHUNK, C), jnp.int32),
            pltpu.VMEM((NCHUNK, C), jnp.int32),
            pltpu.VMEM((NCHUNK, C), jnp.int32),
            pltpu.VMEM((NCHUNK, C), jnp.int32),
            pltpu.VMEM((C, D), jnp.float32),
            pltpu.VMEM((C, D), jnp.float32),
            pltpu.VMEM((C, D), jnp.float32),
            pltpu.VMEM((C, D), jnp.float32),
            pltpu.VMEM((C, D), jnp.float32),
            pltpu.VMEM((C, D), jnp.float32),
            pltpu.VMEM_SHARED((TAB, D), jnp.float32),
            pltpu.SemaphoreType.DMA,
            pltpu.SemaphoreType.DMA,
            pltpu.SemaphoreType.DMA,
        ],
    )
    def k(row_hbm, col_hbm, dir_hbm, tab_hbm, out_hbm,
          ri_v, ci_v, di_v, cd_v, a0_v, a1_v, b0_v, b1_v, o0_v, o1_v,
          sh_tab, gsem, wsem, ssem):
        cid = lax.axis_index("c")
        sid = lax.axis_index("s")
        wid = sid * NC + cid
        chunk0 = wid * NCHUNK

        scp = pltpu.async_copy(tab_hbm.at[pl.ds(sid * STG, STG)],
                               sh_tab.at[pl.ds(sid * STG, STG)], ssem)

        pltpu.sync_copy(row_hbm.at[pl.ds(chunk0, NCHUNK)], ri_v)
        pltpu.sync_copy(col_hbm.at[pl.ds(chunk0, NCHUNK)], ci_v)
        pltpu.sync_copy(dir_hbm.at[pl.ds(chunk0, NCHUNK)], di_v)

        @pl.loop(0, NCHUNK)
        def _(r):
            for j in range(0, C, LANES):
                s = pl.ds(j, LANES)
                cd_v[r, s] = (di_v[r, s] + 1) * L + ci_v[r, s]

        scp.wait()
        plsc.subcore_barrier()

        abufs = (a0_v, a1_v)
        bbufs = (b0_v, b1_v)
        obufs = (o0_v, o1_v)

        def fire(ch):
            a = pltpu.async_copy(sh_tab.at[ri_v.at[ch]], abufs[ch % 2], gsem)
            b = pltpu.async_copy(sh_tab.at[cd_v.at[ch]], bbufs[ch % 2], gsem)
            return a, b

        H = C // 2
        gcp = [fire(0), fire(1)]
        wcp = [None, None]
        for ch in range(NCHUNK):
            gcp[ch % 2][0].wait()
            gcp[ch % 2][1].wait()
            if wcp[ch % 2] is not None:
                wcp[ch % 2][0].wait()
                wcp[ch % 2][1].wait()
            a_v, b_v, o_v = abufs[ch % 2], bbufs[ch % 2], obufs[ch % 2]

            @pl.loop(0, H)
            def _(r):
                for j in range(0, D, LANES):
                    s = pl.ds(j, LANES)
                    o_v[r, s] = a_v[r, s] + b_v[r, s]

            w1 = pltpu.async_copy(
                o_v.at[pl.ds(0, H)],
                out_hbm.at[pl.ds((chunk0 + ch) * C, H)], wsem)

            @pl.loop(H, C)
            def _(r):
                for j in range(0, D, LANES):
                    s = pl.ds(j, LANES)
                    o_v[r, s] = a_v[r, s] + b_v[r, s]

            w2 = pltpu.async_copy(
                o_v.at[pl.ds(H, H)],
                out_hbm.at[pl.ds((chunk0 + ch) * C + H, H)], wsem)
            if ch + 2 < NCHUNK:
                gcp[ch % 2] = fire(ch + 2)
            wcp[ch % 2] = (w1, w2)
        wcp[0][0].wait()
        wcp[0][1].wait()
        wcp[1][0].wait()
        wcp[1][1].wait()

    return k(row2, col2, dir2, tab)
